# baseline (speedup 1.0000x reference)
import functools

import jax
import jax.numpy as jnp
from jax import lax
from jax.experimental import pallas as pl
from jax.experimental.pallas import tpu as pltpu
from jax.experimental.pallas import tpu_sc as plsc

N = 10000
E = 320000
D = 128
L = 3

NC = 2
NS = 16
NW = NC * NS
EPW = E // NW
K = 128
EPWP = 10240
NCH = EPWP // K
NPAD = 10240
SENT = 10239
DH = D // 2
RPT = NPAD // NS

_mesh = plsc.VectorSubcoreMesh(core_axis_name="c", subcore_axis_name="s")


@functools.partial(
    pl.kernel,
    out_type=jax.ShapeDtypeStruct((NC, NPAD), jnp.float32),
    mesh=_mesh,
    scratch_types=[
        pltpu.VMEM((NCH, K), jnp.int32),
        pltpu.VMEM((K,), jnp.float32),
        pltpu.VMEM_SHARED((NPAD,), jnp.float32),
    ],
)
def _deg_kernel(dst_hbm, ones_hbm, zvec_hbm, deg_out, dst_loc, ones_loc, deg_sm):
    c = lax.axis_index("c")
    s = lax.axis_index("s")
    wid = c * NS + s
    pltpu.sync_copy(zvec_hbm, deg_sm.at[pl.ds(s * RPT, RPT)])
    pltpu.sync_copy(dst_hbm.at[wid], dst_loc)
    pltpu.sync_copy(ones_hbm, ones_loc)
    plsc.subcore_barrier()

    def body(j, carry):
        pltpu.sync_copy(ones_loc, deg_sm.at[dst_loc.at[j]], add=True)
        return carry

    lax.fori_loop(0, NCH, body, 0)
    plsc.subcore_barrier()
    pltpu.sync_copy(deg_sm.at[pl.ds(s * RPT, RPT)],
                    deg_out.at[c].at[pl.ds(s * RPT, RPT)])


@functools.partial(
    pl.kernel,
    out_type=jax.ShapeDtypeStruct((NC, NPAD, D), jnp.float32),
    mesh=_mesh,
    scratch_types=[
        pltpu.VMEM((2, NCH, K), jnp.int32),
        pltpu.VMEM((K, D), jnp.float32),
        pltpu.VMEM_SHARED((NPAD, D), jnp.float32),
    ],
)
def _prop_kernel(edge_hbm, z_hbm, zrows_hbm, acc_out, eidx, gbuf, acc_sm):
    c = lax.axis_index("c")
    s = lax.axis_index("s")
    wid = c * NS + s
    pltpu.sync_copy(zrows_hbm, acc_sm.at[pl.ds(s * RPT, RPT)])
    pltpu.sync_copy(edge_hbm.at[wid], eidx)
    plsc.subcore_barrier()

    def body(j, carry):
        pltpu.sync_copy(z_hbm.at[eidx.at[0, j]], gbuf)
        pltpu.sync_copy(gbuf, acc_sm.at[eidx.at[1, j]], add=True)
        return carry

    lax.fori_loop(0, NCH, body, 0)
    plsc.subcore_barrier()
    pltpu.sync_copy(acc_sm.at[pl.ds(s * RPT, RPT)],
                    acc_out.at[c].at[pl.ds(s * RPT, RPT)])


def _dinv_body(deg_ref, o_ref):
    d = deg_ref[0] + deg_ref[1]
    safe = jnp.where(d > 0, d, 1.0)
    o_ref[...] = jnp.where(d > 0, lax.rsqrt(safe), 0.0)


def _prep_body(dinv_ref, x_ref, z_ref):
    z_ref[...] = dinv_ref[...] * x_ref[...]


def _comb_body(acc_ref, dinv_ref, sprev_ref, snew_ref, z_ref):
    sblk = acc_ref[0] + acc_ref[1]
    snew_ref[...] = sprev_ref[...] + sblk
    dv = dinv_ref[...]
    z_ref[...] = dv * dv * sblk


def _comb_final_body(acc_ref, dinv_ref, sprev_ref, x_ref, o_ref):
    stot = sprev_ref[...] + acc_ref[0] + acc_ref[1]
    o_ref[...] = (x_ref[...] + dinv_ref[...] * stot) * 0.25


_RB = 1024
_GRID = NPAD // _RB
_row_spec = pl.BlockSpec((_RB, D), lambda i: (i, 0))
_acc_spec = pl.BlockSpec((NC, _RB, D), lambda i: (0, i, 0))
_sds = lambda: jax.ShapeDtypeStruct((NPAD, D), jnp.float32)

_prep_call = pl.pallas_call(
    _prep_body, grid=(_GRID,), out_shape=_sds(),
    in_specs=[_row_spec, _row_spec], out_specs=_row_spec)

_comb_call = pl.pallas_call(
    _comb_body, grid=(_GRID,), out_shape=(_sds(), _sds()),
    in_specs=[_acc_spec, _row_spec, _row_spec],
    out_specs=(_row_spec, _row_spec))

_comb_final_call = pl.pallas_call(
    _comb_final_body, grid=(_GRID,), out_shape=_sds(),
    in_specs=[_acc_spec, _row_spec, _row_spec, _row_spec],
    out_specs=_row_spec)

_dinv_call = pl.pallas_call(
    _dinv_body, out_shape=jax.ShapeDtypeStruct((NPAD // D, D), jnp.float32))


def kernel(edge_index, emb_weight):
    pad2 = ((0, 0), (0, EPWP - EPW))
    src_rs = jnp.pad(edge_index[0].reshape(NW, EPW), pad2,
                     constant_values=SENT).reshape(NW, NCH, K)
    dst_rs = jnp.pad(edge_index[1].reshape(NW, EPW), pad2,
                     constant_values=SENT).reshape(NW, NCH, K)
    edges_rs = jnp.stack([src_rs, dst_rs], axis=1)
    ones_k = jnp.ones((K,), jnp.float32)
    zvec = jnp.zeros((RPT,), jnp.float32)
    zrows = jnp.zeros((RPT, D), jnp.float32)
    x0 = jnp.concatenate(
        [emb_weight, jnp.zeros((NPAD - N, D), jnp.float32)], axis=0)

    deg2 = _deg_kernel(dst_rs, ones_k, zvec)
    dinv = _dinv_call(deg2.reshape(NC, NPAD // D, D))
    dinvb = jnp.broadcast_to(dinv.reshape(NPAD, 1), (NPAD, D))

    z = _prep_call(dinvb, x0)
    stot = jnp.zeros((NPAD, D), jnp.float32)
    for _ in range(L - 1):
        acc2 = _prop_kernel(edges_rs, z, zrows)
        stot, z = _comb_call(acc2, dinvb, stot)
    acc2 = _prop_kernel(edges_rs, z, zrows)
    out = _comb_final_call(acc2, dinvb, stot, x0)
    return out[:N]


# speedup vs baseline: 2.0192x; 2.0192x over previous
import functools

import jax
import jax.numpy as jnp
from jax import lax
from jax.experimental import pallas as pl
from jax.experimental.pallas import tpu as pltpu
from jax.experimental.pallas import tpu_sc as plsc

N = 10000
E = 320000
D = 128
L = 3

NC = 2
NS = 16
NW = NC * NS
EPW = E // NW
K = 80
EPWP = 10000
NCH = EPWP // K
NPAD = 10240
SENT = 10239
DH = D // 2
RPT = NPAD // NS

_mesh = plsc.VectorSubcoreMesh(core_axis_name="c", subcore_axis_name="s")


@functools.partial(
    pl.kernel,
    out_type=jax.ShapeDtypeStruct((NC, NPAD), jnp.float32),
    mesh=_mesh,
    scratch_types=[
        pltpu.VMEM((NCH, K), jnp.int32),
        pltpu.VMEM((K,), jnp.float32),
        pltpu.VMEM_SHARED((NPAD,), jnp.float32),
    ],
)
def _deg_kernel(dst_hbm, ones_hbm, zvec_hbm, deg_out, dst_loc, ones_loc, deg_sm):
    c = lax.axis_index("c")
    s = lax.axis_index("s")
    wid = c * NS + s
    pltpu.sync_copy(zvec_hbm, deg_sm.at[pl.ds(s * RPT, RPT)])
    pltpu.sync_copy(dst_hbm.at[wid], dst_loc)
    pltpu.sync_copy(ones_hbm, ones_loc)
    plsc.subcore_barrier()

    def body(j, carry):
        pltpu.sync_copy(ones_loc, deg_sm.at[dst_loc.at[j]], add=True)
        return carry

    lax.fori_loop(0, NCH, body, 0)
    plsc.subcore_barrier()
    pltpu.sync_copy(deg_sm.at[pl.ds(s * RPT, RPT)],
                    deg_out.at[c].at[pl.ds(s * RPT, RPT)])


@functools.partial(
    pl.kernel,
    out_type=jax.ShapeDtypeStruct((NC, NPAD, D), jnp.float32),
    mesh=_mesh,
    scratch_types=[
        pltpu.VMEM((2, NCH, K), jnp.int32),
        pltpu.VMEM((K, D), jnp.float32),
        pltpu.VMEM_SHARED((NPAD, D), jnp.float32),
    ],
)
def _prop_kernel(edge_hbm, z_hbm, zrows_hbm, acc_out, eidx, gbuf, acc_sm):
    c = lax.axis_index("c")
    s = lax.axis_index("s")
    wid = c * NS + s
    pltpu.sync_copy(zrows_hbm, acc_sm.at[pl.ds(s * RPT, RPT)])
    pltpu.sync_copy(edge_hbm.at[wid], eidx)
    plsc.subcore_barrier()

    def body(j, carry):
        pltpu.sync_copy(z_hbm.at[eidx.at[0, j]], gbuf)
        pltpu.sync_copy(gbuf, acc_sm.at[eidx.at[1, j]], add=True)
        return carry

    lax.fori_loop(0, NCH, body, 0)
    plsc.subcore_barrier()
    pltpu.sync_copy(acc_sm.at[pl.ds(s * RPT, RPT)],
                    acc_out.at[c].at[pl.ds(s * RPT, RPT)])


def _dinv_body(deg_ref, o_ref):
    d = deg_ref[0] + deg_ref[1]
    safe = jnp.where(d > 0, d, 1.0)
    o_ref[...] = jnp.where(d > 0, lax.rsqrt(safe), 0.0)


def _prep_body(dinv_ref, x_ref, z_ref):
    z_ref[...] = dinv_ref[...] * x_ref[...]


def _comb_body(acc_ref, dinv_ref, sprev_ref, snew_ref, z_ref):
    sblk = acc_ref[0] + acc_ref[1]
    snew_ref[...] = sprev_ref[...] + sblk
    dv = dinv_ref[...]
    z_ref[...] = dv * dv * sblk


def _comb_final_body(acc_ref, dinv_ref, sprev_ref, x_ref, o_ref):
    stot = sprev_ref[...] + acc_ref[0] + acc_ref[1]
    o_ref[...] = (x_ref[...] + dinv_ref[...] * stot) * 0.25


_RB = 1024
_GRID = NPAD // _RB
_row_spec = pl.BlockSpec((_RB, D), lambda i: (i, 0))
_acc_spec = pl.BlockSpec((NC, _RB, D), lambda i: (0, i, 0))
_sds = lambda: jax.ShapeDtypeStruct((NPAD, D), jnp.float32)

_prep_call = pl.pallas_call(
    _prep_body, grid=(_GRID,), out_shape=_sds(),
    in_specs=[_row_spec, _row_spec], out_specs=_row_spec)

_comb_call = pl.pallas_call(
    _comb_body, grid=(_GRID,), out_shape=(_sds(), _sds()),
    in_specs=[_acc_spec, _row_spec, _row_spec],
    out_specs=(_row_spec, _row_spec))

_comb_final_call = pl.pallas_call(
    _comb_final_body, grid=(_GRID,), out_shape=_sds(),
    in_specs=[_acc_spec, _row_spec, _row_spec, _row_spec],
    out_specs=_row_spec)

_dinv_call = pl.pallas_call(
    _dinv_body, out_shape=jax.ShapeDtypeStruct((NPAD // D, D), jnp.float32))


def kernel(edge_index, emb_weight):
    src_rs = edge_index[0].reshape(NW, NCH, K)
    dst_rs = edge_index[1].reshape(NW, NCH, K)
    edges_rs = jnp.stack([src_rs, dst_rs], axis=1)
    ones_k = jnp.ones((K,), jnp.float32)
    zvec = jnp.zeros((RPT,), jnp.float32)
    zrows = jnp.zeros((RPT, D), jnp.float32)
    x0 = jnp.concatenate(
        [emb_weight, jnp.zeros((NPAD - N, D), jnp.float32)], axis=0)

    deg2 = _deg_kernel(dst_rs, ones_k, zvec)
    dinv = _dinv_call(deg2.reshape(NC, NPAD // D, D))
    dinvb = jnp.broadcast_to(dinv.reshape(NPAD, 1), (NPAD, D))

    z = _prep_call(dinvb, x0)
    stot = jnp.zeros((NPAD, D), jnp.float32)
    for _ in range(L - 1):
        acc2 = _prop_kernel(edges_rs, z, zrows)
        stot, z = _comb_call(acc2, dinvb, stot)
    acc2 = _prop_kernel(edges_rs, z, zrows)
    out = _comb_final_call(acc2, dinvb, stot, x0)
    return out[:N]


# speedup vs baseline: 2.3002x; 1.1391x over previous
import functools

import jax
import jax.numpy as jnp
from jax import lax
from jax.experimental import pallas as pl
from jax.experimental.pallas import tpu as pltpu
from jax.experimental.pallas import tpu_sc as plsc

N = 10000
E = 320000
D = 128
L = 3

NC = 2
NS = 16
NW = NC * NS
EPW = E // NW
K = 128
EPWP = 10240
NCH = EPWP // K
NPAD = 10240
SENT = 10239
DH = D // 2
RPT = NPAD // NS

_mesh = plsc.VectorSubcoreMesh(core_axis_name="c", subcore_axis_name="s")


@functools.partial(
    pl.kernel,
    out_type=jax.ShapeDtypeStruct((NC, NPAD), jnp.float32),
    mesh=_mesh,
    scratch_types=[
        pltpu.VMEM((NCH, K), jnp.int32),
        pltpu.VMEM((K,), jnp.float32),
        pltpu.VMEM_SHARED((NPAD,), jnp.float32),
    ],
)
def _deg_kernel(dst_hbm, ones_hbm, zvec_hbm, deg_out, dst_loc, ones_loc, deg_sm):
    c = lax.axis_index("c")
    s = lax.axis_index("s")
    wid = c * NS + s
    pltpu.sync_copy(zvec_hbm, deg_sm.at[pl.ds(s * RPT, RPT)])
    pltpu.sync_copy(dst_hbm.at[wid], dst_loc)
    pltpu.sync_copy(ones_hbm, ones_loc)
    plsc.subcore_barrier()

    def body(j, carry):
        pltpu.sync_copy(ones_loc, deg_sm.at[dst_loc.at[j]], add=True)
        return carry

    lax.fori_loop(0, NCH, body, 0)
    plsc.subcore_barrier()
    pltpu.sync_copy(deg_sm.at[pl.ds(s * RPT, RPT)],
                    deg_out.at[c].at[pl.ds(s * RPT, RPT)])


@functools.partial(
    pl.kernel,
    out_type=jax.ShapeDtypeStruct((NC, NPAD, D), jnp.float32),
    mesh=_mesh,
    scratch_types=[
        pltpu.VMEM((2, NCH, K), jnp.int32),
        pltpu.VMEM((K, D), jnp.float32),
        pltpu.VMEM_SHARED((NPAD, D), jnp.float32),
    ],
)
def _prop_kernel(edge_hbm, z_hbm, zrows_hbm, acc_out, eidx, gbuf, acc_sm):
    c = lax.axis_index("c")
    s = lax.axis_index("s")
    wid = c * NS + s
    pltpu.sync_copy(zrows_hbm, acc_sm.at[pl.ds(s * RPT, RPT)])
    pltpu.sync_copy(edge_hbm.at[wid], eidx)
    plsc.subcore_barrier()

    def body(j, carry):
        pltpu.sync_copy(z_hbm.at[eidx.at[0, j]], gbuf)
        pltpu.sync_copy(gbuf, acc_sm.at[eidx.at[1, j]], add=True)
        return carry

    lax.fori_loop(0, NCH, body, 0)
    plsc.subcore_barrier()
    pltpu.sync_copy(acc_sm.at[pl.ds(s * RPT, RPT)],
                    acc_out.at[c].at[pl.ds(s * RPT, RPT)])


def _dinv_body(deg_ref, o_ref):
    d = deg_ref[0] + deg_ref[1]
    safe = jnp.where(d > 0, d, 1.0)
    o_ref[...] = jnp.where(d > 0, lax.rsqrt(safe), 0.0)


def _prep_body(dinv_ref, x_ref, z_ref):
    z_ref[...] = dinv_ref[...] * x_ref[...]


def _comb_body(acc_ref, dinv_ref, sprev_ref, snew_ref, z_ref):
    sblk = acc_ref[0] + acc_ref[1]
    snew_ref[...] = sprev_ref[...] + sblk
    dv = dinv_ref[...]
    z_ref[...] = dv * dv * sblk


def _comb_final_body(acc_ref, dinv_ref, sprev_ref, x_ref, o_ref):
    stot = sprev_ref[...] + acc_ref[0] + acc_ref[1]
    o_ref[...] = (x_ref[...] + dinv_ref[...] * stot) * 0.25


_RB = 1024
_GRID = NPAD // _RB
_row_spec = pl.BlockSpec((_RB, D), lambda i: (i, 0))
_acc_spec = pl.BlockSpec((NC, _RB, D), lambda i: (0, i, 0))
_sds = lambda: jax.ShapeDtypeStruct((NPAD, D), jnp.float32)

_prep_call = pl.pallas_call(
    _prep_body, grid=(_GRID,), out_shape=_sds(),
    in_specs=[_row_spec, _row_spec], out_specs=_row_spec)

_comb_call = pl.pallas_call(
    _comb_body, grid=(_GRID,), out_shape=(_sds(), _sds()),
    in_specs=[_acc_spec, _row_spec, _row_spec],
    out_specs=(_row_spec, _row_spec))

_comb_final_call = pl.pallas_call(
    _comb_final_body, grid=(_GRID,), out_shape=_sds(),
    in_specs=[_acc_spec, _row_spec, _row_spec, _row_spec],
    out_specs=_row_spec)

_dinv_call = pl.pallas_call(
    _dinv_body, out_shape=jax.ShapeDtypeStruct((NPAD // D, D), jnp.float32))


def kernel(edge_index, emb_weight):
    npad_e = EPWP - EPW
    lanes = jnp.arange(npad_e, dtype=jnp.int32)[None, :]
    wids = jnp.arange(NW, dtype=jnp.int32)[:, None]
    pad_src = N + (lanes * 17 + wids * 31) % (NPAD - N)
    pad_dst = N + (lanes * 23 + wids * 41) % (NPAD - N)
    src_rs = jnp.concatenate(
        [edge_index[0].reshape(NW, EPW), pad_src], axis=1).reshape(NW, NCH, K)
    dst_rs = jnp.concatenate(
        [edge_index[1].reshape(NW, EPW), pad_dst], axis=1).reshape(NW, NCH, K)
    edges_rs = jnp.stack([src_rs, dst_rs], axis=1)
    ones_k = jnp.ones((K,), jnp.float32)
    zvec = jnp.zeros((RPT,), jnp.float32)
    zrows = jnp.zeros((RPT, D), jnp.float32)
    x0 = jnp.concatenate(
        [emb_weight, jnp.zeros((NPAD - N, D), jnp.float32)], axis=0)

    deg2 = _deg_kernel(dst_rs, ones_k, zvec)
    dinv = _dinv_call(deg2.reshape(NC, NPAD // D, D))
    dinvb = jnp.broadcast_to(dinv.reshape(NPAD, 1), (NPAD, D))

    z = _prep_call(dinvb, x0)
    stot = jnp.zeros((NPAD, D), jnp.float32)
    for _ in range(L - 1):
        acc2 = _prop_kernel(edges_rs, z, zrows)
        stot, z = _comb_call(acc2, dinvb, stot)
    acc2 = _prop_kernel(edges_rs, z, zrows)
    out = _comb_final_call(acc2, dinvb, stot, x0)
    return out[:N]


# speedup vs baseline: 2.3168x; 1.0072x over previous
import functools

import jax
import jax.numpy as jnp
from jax import lax
from jax.experimental import pallas as pl
from jax.experimental.pallas import tpu as pltpu
from jax.experimental.pallas import tpu_sc as plsc

N = 10000
E = 320000
D = 128
L = 3

NC = 2
NS = 16
NW = NC * NS
EPW = E // NW
K = 128
EPWP = 10240
NCH = EPWP // K
NPAD = 10240
SENT = 10239
DH = D // 2
RPT = NPAD // NS

_mesh = plsc.VectorSubcoreMesh(core_axis_name="c", subcore_axis_name="s")


@functools.partial(
    pl.kernel,
    out_type=jax.ShapeDtypeStruct((NC, NPAD), jnp.float32),
    mesh=_mesh,
    scratch_types=[
        pltpu.VMEM((NCH, K), jnp.int32),
        pltpu.VMEM((K,), jnp.float32),
        pltpu.VMEM_SHARED((NPAD,), jnp.float32),
    ],
)
def _deg_kernel(dst_hbm, ones_hbm, zvec_hbm, deg_out, dst_loc, ones_loc, deg_sm):
    c = lax.axis_index("c")
    s = lax.axis_index("s")
    wid = c * NS + s
    pltpu.sync_copy(zvec_hbm, deg_sm.at[pl.ds(s * RPT, RPT)])
    pltpu.sync_copy(dst_hbm.at[wid], dst_loc)
    pltpu.sync_copy(ones_hbm, ones_loc)
    plsc.subcore_barrier()

    def body(j, carry):
        pltpu.sync_copy(ones_loc, deg_sm.at[dst_loc.at[j]], add=True)
        return carry

    lax.fori_loop(0, NCH, body, 0)
    plsc.subcore_barrier()
    pltpu.sync_copy(deg_sm.at[pl.ds(s * RPT, RPT)],
                    deg_out.at[c].at[pl.ds(s * RPT, RPT)])


@functools.partial(
    pl.kernel,
    out_type=jax.ShapeDtypeStruct((NC, NPAD, D), jnp.float32),
    mesh=_mesh,
    scratch_types=[
        pltpu.VMEM((2, NCH, K), jnp.int32),
        pltpu.VMEM((K, D), jnp.float32),
        pltpu.VMEM_SHARED((NPAD, D), jnp.float32),
    ],
)
def _prop_kernel(edge_hbm, z_hbm, zrows_hbm, acc_out, eidx, gbuf, acc_sm):
    c = lax.axis_index("c")
    s = lax.axis_index("s")
    wid = c * NS + s
    pltpu.sync_copy(zrows_hbm, acc_sm.at[pl.ds(s * RPT, RPT)])
    pltpu.sync_copy(edge_hbm.at[wid], eidx)
    plsc.subcore_barrier()

    def body(j, carry):
        pltpu.sync_copy(z_hbm.at[eidx.at[0, j]], gbuf)
        pltpu.sync_copy(gbuf, acc_sm.at[eidx.at[1, j]], add=True)
        return carry

    lax.fori_loop(0, NCH, body, 0)
    plsc.subcore_barrier()
    pltpu.sync_copy(acc_sm.at[pl.ds(s * RPT, RPT)],
                    acc_out.at[c].at[pl.ds(s * RPT, RPT)])


def _dinv_body(deg_ref, o_ref):
    d = deg_ref[0] + deg_ref[1]
    safe = jnp.where(d > 0, d, 1.0)
    o_ref[...] = jnp.where(d > 0, lax.rsqrt(safe), 0.0)


def _prep_body(dinv_ref, x_ref, z_ref):
    z_ref[...] = dinv_ref[...] * x_ref[...]


def _comb_body(acc_ref, dinv_ref, z_ref):
    sblk = acc_ref[0] + acc_ref[1]
    dv = dinv_ref[...]
    z_ref[...] = dv * dv * sblk


def _final_body(a1_ref, a2_ref, a3_ref, dinv_ref, x_ref, o_ref):
    stot = (a1_ref[0] + a1_ref[1] + a2_ref[0] + a2_ref[1]
            + a3_ref[0] + a3_ref[1])
    o_ref[...] = (x_ref[...] + dinv_ref[...] * stot) * 0.25


_RB = 1024
_GRID = NPAD // _RB
_row_spec = pl.BlockSpec((_RB, D), lambda i: (i, 0))
_acc_spec = pl.BlockSpec((NC, _RB, D), lambda i: (0, i, 0))
_sds = lambda: jax.ShapeDtypeStruct((NPAD, D), jnp.float32)

_prep_call = pl.pallas_call(
    _prep_body, grid=(_GRID,), out_shape=_sds(),
    in_specs=[_row_spec, _row_spec], out_specs=_row_spec)

_comb_call = pl.pallas_call(
    _comb_body, grid=(_GRID,), out_shape=_sds(),
    in_specs=[_acc_spec, _row_spec], out_specs=_row_spec)

_final_call = pl.pallas_call(
    _final_body, grid=(_GRID,), out_shape=_sds(),
    in_specs=[_acc_spec, _acc_spec, _acc_spec, _row_spec, _row_spec],
    out_specs=_row_spec)

_dinv_call = pl.pallas_call(
    _dinv_body, out_shape=jax.ShapeDtypeStruct((NPAD // D, D), jnp.float32))


def kernel(edge_index, emb_weight):
    npad_e = EPWP - EPW
    lanes = jnp.arange(npad_e, dtype=jnp.int32)[None, :]
    wids = jnp.arange(NW, dtype=jnp.int32)[:, None]
    pad_src = N + (lanes * 17 + wids * 31) % (NPAD - N)
    pad_dst = N + (lanes * 23 + wids * 41) % (NPAD - N)
    src_rs = jnp.concatenate(
        [edge_index[0].reshape(NW, EPW), pad_src], axis=1).reshape(NW, NCH, K)
    dst_rs = jnp.concatenate(
        [edge_index[1].reshape(NW, EPW), pad_dst], axis=1).reshape(NW, NCH, K)
    edges_rs = jnp.stack([src_rs, dst_rs], axis=1)
    ones_k = jnp.ones((K,), jnp.float32)
    zvec = jnp.zeros((RPT,), jnp.float32)
    zrows = jnp.zeros((RPT, D), jnp.float32)
    x0 = jnp.concatenate(
        [emb_weight, jnp.zeros((NPAD - N, D), jnp.float32)], axis=0)

    deg2 = _deg_kernel(dst_rs, ones_k, zvec)
    dinv = _dinv_call(deg2.reshape(NC, NPAD // D, D))
    dinvb = jnp.broadcast_to(dinv.reshape(NPAD, 1), (NPAD, D))

    z = _prep_call(dinvb, x0)
    accs = []
    for _ in range(L):
        acc2 = _prop_kernel(edges_rs, z, zrows)
        accs.append(acc2)
        if len(accs) < L:
            z = _comb_call(acc2, dinvb)
    out = _final_call(accs[0], accs[1], accs[2], dinvb, x0)
    return out[:N]


# speedup vs baseline: 2.3410x; 1.0104x over previous
import functools

import jax
import jax.numpy as jnp
from jax import lax
from jax.experimental import pallas as pl
from jax.experimental.pallas import tpu as pltpu
from jax.experimental.pallas import tpu_sc as plsc

N = 10000
E = 320000
D = 128
L = 3

NC = 2
NS = 16
NW = NC * NS
EPW = E // NW
K = 125
EPWP = 10000
NCH = EPWP // K
NPAD = 10240
SENT = 10239
DH = D // 2
RPT = NPAD // NS

_mesh = plsc.VectorSubcoreMesh(core_axis_name="c", subcore_axis_name="s")


@functools.partial(
    pl.kernel,
    out_type=jax.ShapeDtypeStruct((NC, NPAD), jnp.float32),
    mesh=_mesh,
    scratch_types=[
        pltpu.VMEM((NCH, K), jnp.int32),
        pltpu.VMEM((K,), jnp.float32),
        pltpu.VMEM_SHARED((NPAD,), jnp.float32),
    ],
)
def _deg_kernel(dst_hbm, ones_hbm, zvec_hbm, deg_out, dst_loc, ones_loc, deg_sm):
    c = lax.axis_index("c")
    s = lax.axis_index("s")
    wid = c * NS + s
    pltpu.sync_copy(zvec_hbm, deg_sm.at[pl.ds(s * RPT, RPT)])
    pltpu.sync_copy(dst_hbm.at[wid], dst_loc)
    pltpu.sync_copy(ones_hbm, ones_loc)
    plsc.subcore_barrier()

    def body(j, carry):
        pltpu.sync_copy(ones_loc, deg_sm.at[dst_loc.at[j]], add=True)
        return carry

    lax.fori_loop(0, NCH, body, 0)
    plsc.subcore_barrier()
    pltpu.sync_copy(deg_sm.at[pl.ds(s * RPT, RPT)],
                    deg_out.at[c].at[pl.ds(s * RPT, RPT)])


@functools.partial(
    pl.kernel,
    out_type=jax.ShapeDtypeStruct((NC, NPAD, D), jnp.float32),
    mesh=_mesh,
    scratch_types=[
        pltpu.VMEM((2, NCH, K), jnp.int32),
        pltpu.VMEM((K, D), jnp.float32),
        pltpu.VMEM_SHARED((NPAD, D), jnp.float32),
    ],
)
def _prop_kernel(edge_hbm, z_hbm, zrows_hbm, acc_out, eidx, gbuf, acc_sm):
    c = lax.axis_index("c")
    s = lax.axis_index("s")
    wid = c * NS + s
    pltpu.sync_copy(zrows_hbm, acc_sm.at[pl.ds(s * RPT, RPT)])
    pltpu.sync_copy(edge_hbm.at[wid], eidx)
    plsc.subcore_barrier()

    def body(j, carry):
        pltpu.sync_copy(z_hbm.at[eidx.at[0, j]], gbuf)
        pltpu.sync_copy(gbuf, acc_sm.at[eidx.at[1, j]], add=True)
        return carry

    lax.fori_loop(0, NCH, body, 0)
    plsc.subcore_barrier()
    pltpu.sync_copy(acc_sm.at[pl.ds(s * RPT, RPT)],
                    acc_out.at[c].at[pl.ds(s * RPT, RPT)])


def _dinv_body(deg_ref, o_ref):
    d = deg_ref[0] + deg_ref[1]
    safe = jnp.where(d > 0, d, 1.0)
    o_ref[...] = jnp.where(d > 0, lax.rsqrt(safe), 0.0)


def _prep_body(dinv_ref, x_ref, z_ref):
    z_ref[...] = dinv_ref[...] * x_ref[...]


def _comb_body(acc_ref, dinv_ref, z_ref):
    sblk = acc_ref[0] + acc_ref[1]
    dv = dinv_ref[...]
    z_ref[...] = dv * dv * sblk


def _final_body(a1_ref, a2_ref, a3_ref, dinv_ref, x_ref, o_ref):
    stot = (a1_ref[0] + a1_ref[1] + a2_ref[0] + a2_ref[1]
            + a3_ref[0] + a3_ref[1])
    o_ref[...] = (x_ref[...] + dinv_ref[...] * stot) * 0.25


_RB = 1024
_GRID = NPAD // _RB
_row_spec = pl.BlockSpec((_RB, D), lambda i: (i, 0))
_acc_spec = pl.BlockSpec((NC, _RB, D), lambda i: (0, i, 0))
_sds = lambda: jax.ShapeDtypeStruct((NPAD, D), jnp.float32)

_prep_call = pl.pallas_call(
    _prep_body, grid=(_GRID,), out_shape=_sds(),
    in_specs=[_row_spec, _row_spec], out_specs=_row_spec)

_comb_call = pl.pallas_call(
    _comb_body, grid=(_GRID,), out_shape=_sds(),
    in_specs=[_acc_spec, _row_spec], out_specs=_row_spec)

_final_call = pl.pallas_call(
    _final_body, grid=(_GRID,), out_shape=_sds(),
    in_specs=[_acc_spec, _acc_spec, _acc_spec, _row_spec, _row_spec],
    out_specs=_row_spec)

_dinv_call = pl.pallas_call(
    _dinv_body, out_shape=jax.ShapeDtypeStruct((NPAD // D, D), jnp.float32))


def kernel(edge_index, emb_weight):
    src_rs = edge_index[0].reshape(NW, NCH, K)
    dst_rs = edge_index[1].reshape(NW, NCH, K)
    edges_rs = jnp.stack([src_rs, dst_rs], axis=1)
    ones_k = jnp.ones((K,), jnp.float32)
    zvec = jnp.zeros((RPT,), jnp.float32)
    zrows = jnp.zeros((RPT, D), jnp.float32)
    x0 = jnp.concatenate(
        [emb_weight, jnp.zeros((NPAD - N, D), jnp.float32)], axis=0)

    deg2 = _deg_kernel(dst_rs, ones_k, zvec)
    dinv = _dinv_call(deg2.reshape(NC, NPAD // D, D))
    dinvb = jnp.broadcast_to(dinv.reshape(NPAD, 1), (NPAD, D))

    z = _prep_call(dinvb, x0)
    accs = []
    for _ in range(L):
        acc2 = _prop_kernel(edges_rs, z, zrows)
        accs.append(acc2)
        if len(accs) < L:
            z = _comb_call(acc2, dinvb)
    out = _final_call(accs[0], accs[1], accs[2], dinvb, x0)
    return out[:N]
